# baseline (speedup 1.0000x reference)
import functools

import jax
import jax.numpy as jnp
from jax import lax
from jax.experimental import pallas as pl
from jax.experimental.pallas import tpu as pltpu
from jax.experimental.pallas import tpu_sc as plsc

N_NODES = 50000
N_EDGES = 800000
N_MOL = 512
D = 64
H = 32
NC = 2
NS = 16
LANES = 16
NTYPES = 128
NBOND = 4

EPW = 50688
NE_PAD = EPW * NS
BLK = 128
NBLK = EPW // BLK
NBUF = 6
GAHEAD = 4
NGRP = NBLK // NBUF

NPT = 3200
NN_PAD = NPT * NS
DUMMY_NODE = N_NODES
MOL_ROWS = 640
DUMMY_MOL = N_MOL

_MESH = plsc.VectorSubcoreMesh(core_axis_name="c", subcore_axis_name="s")
_SC_PARAMS = pltpu.CompilerParams(use_tc_tiling_on_sc=False)


def _edge_pass_body(e4_hbm, dst2_hbm, tab_hbm, zrows_hbm, out_hbm,
                    idx_v, dst_v, rows_v, acc_sh,
                    gs0, gs1, gs2, gs3, gs4, gs5,
                    ss0, ss1, ss2, ss3, ss4, ss5):
    c = lax.axis_index("c")
    s = lax.axis_index("s")
    gs = (gs0, gs1, gs2, gs3, gs4, gs5)
    ss = (ss0, ss1, ss2, ss3, ss4, ss5)

    pltpu.sync_copy(zrows_hbm, acc_sh.at[pl.ds(s * NPT, NPT)])
    plsc.subcore_barrier()

    tab_off = c * (NBOND * NN_PAD)
    GSTEP = NBUF * BLK

    def load_group(q, g):
        e0 = s * EPW + g * GSTEP
        pltpu.sync_copy(e4_hbm.at[pl.ds(e0, GSTEP)], idx_v.at[q])

        def add_off(k, carry):
            sl = pl.ds(k * LANES, LANES)
            idx_v[q, sl] = idx_v[q, sl] + tab_off
            return carry

        lax.fori_loop(0, GSTEP // LANES, add_off, 0)
        pltpu.sync_copy(dst2_hbm.at[pl.ds(s * NBLK + g * NBUF, NBUF)],
                        dst_v.at[q])

    def fire_gather(q, brow, slot):
        pltpu.async_copy(tab_hbm.at[idx_v.at[q, pl.ds(brow * BLK, BLK)]],
                         rows_v.at[slot], gs[slot])

    def wait_gather(slot):
        pltpu.make_async_copy(tab_hbm.at[idx_v.at[0, pl.ds(0, BLK)]],
                              rows_v.at[slot], gs[slot]).wait()

    def wait_scatter(slot):
        pltpu.make_async_copy(rows_v.at[slot], acc_sh.at[dst_v.at[0, 0]],
                              ss[slot]).wait()

    load_group(0, 0)
    for b in range(GAHEAD):
        fire_gather(0, b, b)

    def group(g, carry):
        p = lax.rem(g, 2)

        @pl.when(g > 0)
        def _():
            wait_scatter(NBUF - 2)
            wait_scatter(NBUF - 1)

        @pl.when(g < NGRP - 1)
        def _():
            load_group(1 - p, g + 1)

        for b in range(NBUF):
            wait_gather(b)
            pltpu.async_copy(rows_v.at[b], acc_sh.at[dst_v.at[p, b]],
                             ss[b], add=True)
            if b >= 2:
                wait_scatter(b - 2)
            nslot = (b + GAHEAD) % NBUF
            if b < NBUF - GAHEAD:
                fire_gather(p, b + GAHEAD, nslot)
            else:
                @pl.when(g < NGRP - 1)
                def _():
                    fire_gather(1 - p, b + GAHEAD - NBUF, nslot)
        return carry

    lax.fori_loop(0, NGRP, group, 0)
    wait_scatter((NBLK - 2) % NBUF)
    wait_scatter((NBLK - 1) % NBUF)

    plsc.subcore_barrier()
    pltpu.sync_copy(acc_sh.at[pl.ds(s * NPT, NPT)],
                    out_hbm.at[pl.ds(c * NN_PAD + s * NPT, NPT)])


def _mol_body(node_hbm, batch2_hbm, zmol_hbm, out_hbm,
              bt_v, nrows_v, acc_sh, sem):
    c = lax.axis_index("c")
    s = lax.axis_index("s")
    rpt = MOL_ROWS // NS

    pltpu.sync_copy(zmol_hbm, acc_sh.at[pl.ds(s * rpt, rpt)])
    pltpu.sync_copy(batch2_hbm.at[pl.ds(s * (NPT // 128), NPT // 128)], bt_v)
    plsc.subcore_barrier()

    def block(q, carry):
        r0 = c * NN_PAD + s * NPT + q * 128
        pltpu.sync_copy(node_hbm.at[pl.ds(r0, 128)], nrows_v)
        pltpu.sync_copy(nrows_v, acc_sh.at[bt_v.at[q]], add=True)
        return carry

    lax.fori_loop(0, NPT // 128, block, 0)
    plsc.subcore_barrier()

    rows_out = N_MOL // NS
    pltpu.sync_copy(acc_sh.at[pl.ds(s * rows_out, rows_out)],
                    out_hbm.at[c, pl.ds(s * rows_out, rows_out)])


BLKN = 1024
_NBLK_N = NN_PAD // BLKN


def _tc_table_init_body(x_ref, tw_ref, bw_ref, out_ref):
    ks = lax.broadcasted_iota(jnp.int32, (BLKN, NTYPES), 1).astype(jnp.float32)
    onehot = (x_ref[...] == ks).astype(jnp.float32)
    res = jnp.dot(onehot, tw_ref[0], preferred_element_type=jnp.float32)
    out_ref[...] = jnp.maximum(res + bw_ref[0], 0.0)


def _tc_table_pass_body(x_ref, a0_ref, a1_ref, tu_ref, w_ref, bu_ref,
                        out_ref):
    ks = lax.broadcasted_iota(jnp.int32, (BLKN, NTYPES), 1).astype(jnp.float32)
    onehot = (x_ref[...] == ks).astype(jnp.float32)
    emb = jnp.dot(onehot, tu_ref[0], preferred_element_type=jnp.float32)
    xfull = jnp.concatenate([a0_ref[...], a1_ref[...]], axis=1)
    res = jnp.dot(xfull, w_ref[0], preferred_element_type=jnp.float32) + emb
    out_ref[...] = jnp.maximum(res + bu_ref[0], 0.0)


def _tc_mlp_body(mol_ref, w1_ref, b1_ref, w2_ref, b2_ref, out_ref):
    h = jnp.maximum(jnp.dot(mol_ref[...], w1_ref[...],
                            preferred_element_type=jnp.float32) + b1_ref[...],
                    0.0)
    out_ref[...] = jnp.dot(h, w2_ref[...],
                           preferred_element_type=jnp.float32) + b2_ref[...]


_edge_pass = functools.partial(
    pl.kernel, _edge_pass_body,
    out_type=jax.ShapeDtypeStruct((NC * NN_PAD, H), jnp.float32),
    mesh=_MESH,
    compiler_params=_SC_PARAMS,
    scratch_types=[
        pltpu.VMEM((2, NBUF * BLK), jnp.int32),
        pltpu.VMEM((2, NBUF, BLK), jnp.int32),
        pltpu.VMEM((NBUF, BLK, H), jnp.float32),
        pltpu.VMEM_SHARED((NN_PAD, H), jnp.float32),
    ] + [pltpu.SemaphoreType.DMA] * 12,
)

_mol_reduce = functools.partial(
    pl.kernel, _mol_body,
    out_type=jax.ShapeDtypeStruct((NC, N_MOL, H), jnp.float32),
    mesh=_MESH,
    compiler_params=_SC_PARAMS,
    scratch_types=[
        pltpu.VMEM((NPT // 128, 128), jnp.int32),
        pltpu.VMEM((128, H), jnp.float32),
        pltpu.VMEM_SHARED((MOL_ROWS, H), jnp.float32),
        pltpu.SemaphoreType.DMA,
    ],
)


def _tc_table_init(x2, tw_pad, bw4):
    return pl.pallas_call(
        _tc_table_init_body,
        grid=(NC, NBOND, _NBLK_N),
        in_specs=[
            pl.BlockSpec((BLKN, 1), lambda c, a, i: (i, 0)),
            pl.BlockSpec((1, NTYPES, H), lambda c, a, i: (c, 0, 0)),
            pl.BlockSpec((1, 1, H), lambda c, a, i: (c * NBOND + a, 0, 0)),
        ],
        out_specs=pl.BlockSpec(
            (BLKN, H),
            lambda c, a, i: ((c * NBOND + a) * _NBLK_N + i, 0)),
        out_shape=jax.ShapeDtypeStruct((NC * NBOND * NN_PAD, H), jnp.float32),
    )(x2, tw_pad, bw4)


def _tc_table_pass(x2, agg, tu_pad, w_agg, bu4):
    return pl.pallas_call(
        _tc_table_pass_body,
        grid=(NC, NBOND, _NBLK_N),
        in_specs=[
            pl.BlockSpec((BLKN, 1), lambda c, a, i: (i, 0)),
            pl.BlockSpec((BLKN, H), lambda c, a, i: (i, 0)),
            pl.BlockSpec((BLKN, H), lambda c, a, i: (_NBLK_N + i, 0)),
            pl.BlockSpec((1, NTYPES, H), lambda c, a, i: (c, 0, 0)),
            pl.BlockSpec((1, D, H), lambda c, a, i: (c, 0, 0)),
            pl.BlockSpec((1, 1, H), lambda c, a, i: (c * NBOND + a, 0, 0)),
        ],
        out_specs=pl.BlockSpec(
            (BLKN, H),
            lambda c, a, i: ((c * NBOND + a) * _NBLK_N + i, 0)),
        out_shape=jax.ShapeDtypeStruct((NC * NBOND * NN_PAD, H), jnp.float32),
    )(x2, agg, agg, tu_pad, w_agg, bu4)


def _tc_mlp(mol, w1, b1, w2, b2):
    return pl.pallas_call(
        _tc_mlp_body,
        out_shape=jax.ShapeDtypeStruct((N_MOL, 1), jnp.float32),
    )(mol, w1, b1.reshape(1, -1), w2, b2.reshape(1, 1))


def kernel(x, edge_index, edge_attr, batch, atom_table, bond_table,
           W_init, b_init, W_upd, b_upd, W1, b1, W2, b2):
    A = atom_table.shape[1]
    B = bond_table.shape[1]

    tw_full = jnp.zeros((NTYPES, D), jnp.float32).at[:119].set(
        atom_table @ W_init[:A])
    tu_full = jnp.zeros((NTYPES, D), jnp.float32).at[:119].set(
        atom_table @ W_upd[:A])
    tw_pad = jnp.stack([tw_full[:, :H], tw_full[:, H:]])
    tu_pad = jnp.stack([tu_full[:, :H], tu_full[:, H:]])
    bondW4 = bond_table @ W_init[A:] + b_init
    bondU4 = bond_table @ W_upd[A:A + B] + b_upd
    bw4 = jnp.stack([bondW4[:, :H], bondW4[:, H:]]).reshape(NC * NBOND, 1, H)
    bu4 = jnp.stack([bondU4[:, :H], bondU4[:, H:]]).reshape(NC * NBOND, 1, H)
    w_full = W_upd[A + B:]
    w_agg = jnp.stack([w_full[:, :H], w_full[:, H:]])

    src = edge_index[0]
    dst = edge_index[1]
    pad_e = NE_PAD - N_EDGES
    e4 = jnp.concatenate([edge_attr * NN_PAD + src,
                          jnp.zeros((pad_e,), jnp.int32)])
    dst_p = jnp.concatenate(
        [dst, jnp.full((pad_e,), DUMMY_NODE, jnp.int32)]).reshape(-1, 128)
    batch_p = jnp.concatenate(
        [batch, jnp.full((NN_PAD - N_NODES,), DUMMY_MOL, jnp.int32)]
    ).reshape(-1, 128)
    x2 = jnp.concatenate(
        [x, jnp.zeros((NN_PAD - N_NODES,), jnp.int32)]
    ).astype(jnp.float32).reshape(NN_PAD, 1)

    zrows = jnp.zeros((NPT, H), jnp.float32)
    zmol = jnp.zeros((MOL_ROWS // NS, H), jnp.float32)

    tab = _tc_table_init(x2, tw_pad, bw4)
    agg = _edge_pass()(e4, dst_p, tab, zrows)
    for _ in range(3):
        tab = _tc_table_pass(x2, agg, tu_pad, w_agg, bu4)
        agg = _edge_pass()(e4, dst_p, tab, zrows)

    mol_halves = _mol_reduce()(agg, batch_p, zmol)
    mol = jnp.concatenate([mol_halves[0], mol_halves[1]], axis=1)
    out = _tc_mlp(mol, W1, b1, W2, b2)
    return out[:, 0]


# speedup vs baseline: 1.3959x; 1.3959x over previous
import functools

import jax
import jax.numpy as jnp
from jax import lax
from jax.experimental import pallas as pl
from jax.experimental.pallas import tpu as pltpu
from jax.experimental.pallas import tpu_sc as plsc

N_NODES = 50000
N_EDGES = 800000
N_MOL = 512
D = 64
H = 32
NC = 2
NS = 16
LANES = 16
NTYPES = 128
NBOND = 4

EPW = 50688
NE_PAD = EPW * NS
BLK = 128
NBLK = EPW // BLK
NBUF = 6
GAHEAD = 4
NGRP = NBLK // NBUF

NPT = 3200
NN_PAD = NPT * NS
DUMMY_NODE = N_NODES
MOL_ROWS = 640
DUMMY_MOL = N_MOL

_MESH = plsc.VectorSubcoreMesh(core_axis_name="c", subcore_axis_name="s")
_SC_PARAMS = pltpu.CompilerParams(use_tc_tiling_on_sc=False)


def _edge_pass_body(e4_hbm, dst2_hbm, tab_hbm, zrows_hbm, out_hbm,
                    idx_v, dst_v, rows_v, acc_sh,
                    gs0, gs1, gs2, gs3, gs4, gs5,
                    ss0, ss1, ss2, ss3, ss4, ss5):
    c = lax.axis_index("c")
    s = lax.axis_index("s")
    gs = (gs0, gs1, gs2, gs3, gs4, gs5)
    ss = (ss0, ss1, ss2, ss3, ss4, ss5)

    pltpu.sync_copy(zrows_hbm, acc_sh.at[pl.ds(s * NPT, NPT)])
    plsc.subcore_barrier()

    tab_off = c * (NBOND * NN_PAD)
    GSTEP = NBUF * BLK

    def load_group(q, g):
        e0 = s * EPW + g * GSTEP
        pltpu.sync_copy(e4_hbm.at[pl.ds(e0, GSTEP)], idx_v.at[q])

        def add_off(k, carry):
            sl = pl.ds(k * LANES, LANES)
            idx_v[q, sl] = idx_v[q, sl] + tab_off
            return carry

        lax.fori_loop(0, GSTEP // LANES, add_off, 0)
        pltpu.sync_copy(dst2_hbm.at[pl.ds(s * NBLK + g * NBUF, NBUF)],
                        dst_v.at[q])

    def fire_gather(q, brow, slot):
        pltpu.async_copy(tab_hbm.at[idx_v.at[q, pl.ds(brow * BLK, BLK)]],
                         rows_v.at[slot], gs[slot])

    def wait_gather(slot):
        pltpu.make_async_copy(tab_hbm.at[idx_v.at[0, pl.ds(0, BLK)]],
                              rows_v.at[slot], gs[slot]).wait()

    def wait_scatter(slot):
        pltpu.make_async_copy(rows_v.at[slot], acc_sh.at[dst_v.at[0, 0]],
                              ss[slot]).wait()

    load_group(0, 0)
    for b in range(GAHEAD):
        fire_gather(0, b, b)

    def group(g, carry):
        p = lax.rem(g, 2)

        @pl.when(g > 0)
        def _():
            wait_scatter(NBUF - 2)
            wait_scatter(NBUF - 1)

        @pl.when(g < NGRP - 1)
        def _():
            load_group(1 - p, g + 1)

        for b in range(NBUF):
            wait_gather(b)
            pltpu.async_copy(rows_v.at[b], acc_sh.at[dst_v.at[p, b]],
                             ss[b], add=True)
            if b >= 2:
                wait_scatter(b - 2)
            nslot = (b + GAHEAD) % NBUF
            if b < NBUF - GAHEAD:
                fire_gather(p, b + GAHEAD, nslot)
            else:
                @pl.when(g < NGRP - 1)
                def _():
                    fire_gather(1 - p, b + GAHEAD - NBUF, nslot)
        return carry

    lax.fori_loop(0, NGRP, group, 0)
    wait_scatter((NBLK - 2) % NBUF)
    wait_scatter((NBLK - 1) % NBUF)

    plsc.subcore_barrier()
    pltpu.sync_copy(acc_sh.at[pl.ds(s * NPT, NPT)],
                    out_hbm.at[pl.ds(c * NN_PAD + s * NPT, NPT)])


def _mol_body(node_hbm, batch2_hbm, zmol_hbm, out_hbm,
              bt_v, nrows_v, acc_sh, sem):
    c = lax.axis_index("c")
    s = lax.axis_index("s")
    rpt = MOL_ROWS // NS

    pltpu.sync_copy(zmol_hbm, acc_sh.at[pl.ds(s * rpt, rpt)])
    pltpu.sync_copy(batch2_hbm.at[pl.ds(s * (NPT // 128), NPT // 128)], bt_v)
    plsc.subcore_barrier()

    def block(q, carry):
        r0 = c * NN_PAD + s * NPT + q * 128
        pltpu.sync_copy(node_hbm.at[pl.ds(r0, 128)], nrows_v)
        pltpu.sync_copy(nrows_v, acc_sh.at[bt_v.at[q]], add=True)
        return carry

    lax.fori_loop(0, NPT // 128, block, 0)
    plsc.subcore_barrier()

    rows_out = N_MOL // NS
    pltpu.sync_copy(acc_sh.at[pl.ds(s * rows_out, rows_out)],
                    out_hbm.at[c, pl.ds(s * rows_out, rows_out)])


BLKN = 1024
_NBLK_N = NN_PAD // BLKN


def _tc_table_init_body(x_ref, tw_ref, bw_ref, out_ref):
    ks = lax.broadcasted_iota(jnp.int32, (BLKN, NTYPES), 1).astype(jnp.float32)
    onehot = (x_ref[...] == ks).astype(jnp.float32)
    res = jnp.dot(onehot, tw_ref[0], preferred_element_type=jnp.float32)
    out_ref[...] = jnp.maximum(res[:, None, :] + bw_ref[0][None, :, :], 0.0)


def _tc_table_pass_body(x_ref, a0_ref, a1_ref, tu_ref, w_ref, bu_ref,
                        out_ref):
    ks = lax.broadcasted_iota(jnp.int32, (BLKN, NTYPES), 1).astype(jnp.float32)
    onehot = (x_ref[...] == ks).astype(jnp.float32)
    emb = jnp.dot(onehot, tu_ref[0], preferred_element_type=jnp.float32)
    xfull = jnp.concatenate([a0_ref[...], a1_ref[...]], axis=1)
    res = jnp.dot(xfull, w_ref[0], preferred_element_type=jnp.float32) + emb
    out_ref[...] = jnp.maximum(res[:, None, :] + bu_ref[0][None, :, :], 0.0)


def _tc_mlp_body(mol_ref, w1_ref, b1_ref, w2_ref, b2_ref, out_ref):
    h = jnp.maximum(jnp.dot(mol_ref[...], w1_ref[...],
                            preferred_element_type=jnp.float32) + b1_ref[...],
                    0.0)
    out_ref[...] = jnp.dot(h, w2_ref[...],
                           preferred_element_type=jnp.float32) + b2_ref[...]


_edge_pass = functools.partial(
    pl.kernel, _edge_pass_body,
    out_type=jax.ShapeDtypeStruct((NC * NN_PAD, H), jnp.float32),
    mesh=_MESH,
    compiler_params=_SC_PARAMS,
    scratch_types=[
        pltpu.VMEM((2, NBUF * BLK), jnp.int32),
        pltpu.VMEM((2, NBUF, BLK), jnp.int32),
        pltpu.VMEM((NBUF, BLK, H), jnp.float32),
        pltpu.VMEM_SHARED((NN_PAD, H), jnp.float32),
    ] + [pltpu.SemaphoreType.DMA] * 12,
)

_mol_reduce = functools.partial(
    pl.kernel, _mol_body,
    out_type=jax.ShapeDtypeStruct((NC, N_MOL, H), jnp.float32),
    mesh=_MESH,
    compiler_params=_SC_PARAMS,
    scratch_types=[
        pltpu.VMEM((NPT // 128, 128), jnp.int32),
        pltpu.VMEM((128, H), jnp.float32),
        pltpu.VMEM_SHARED((MOL_ROWS, H), jnp.float32),
        pltpu.SemaphoreType.DMA,
    ],
)


def _tc_table_init(x2, tw_pad, bw4):
    return pl.pallas_call(
        _tc_table_init_body,
        grid=(NC, _NBLK_N),
        in_specs=[
            pl.BlockSpec((BLKN, 1), lambda c, i: (i, 0)),
            pl.BlockSpec((1, NTYPES, H), lambda c, i: (c, 0, 0)),
            pl.BlockSpec((1, NBOND, H), lambda c, i: (c, 0, 0)),
        ],
        out_specs=pl.BlockSpec((BLKN, NBOND, H),
                               lambda c, i: (c * _NBLK_N + i, 0, 0)),
        out_shape=jax.ShapeDtypeStruct((NC * NN_PAD, NBOND, H), jnp.float32),
    )(x2, tw_pad, bw4)


def _tc_table_pass(x2, agg, tu_pad, w_agg, bu4):
    return pl.pallas_call(
        _tc_table_pass_body,
        grid=(NC, _NBLK_N),
        in_specs=[
            pl.BlockSpec((BLKN, 1), lambda c, i: (i, 0)),
            pl.BlockSpec((BLKN, H), lambda c, i: (i, 0)),
            pl.BlockSpec((BLKN, H), lambda c, i: (_NBLK_N + i, 0)),
            pl.BlockSpec((1, NTYPES, H), lambda c, i: (c, 0, 0)),
            pl.BlockSpec((1, D, H), lambda c, i: (c, 0, 0)),
            pl.BlockSpec((1, NBOND, H), lambda c, i: (c, 0, 0)),
        ],
        out_specs=pl.BlockSpec((BLKN, NBOND, H),
                               lambda c, i: (c * _NBLK_N + i, 0, 0)),
        out_shape=jax.ShapeDtypeStruct((NC * NN_PAD, NBOND, H), jnp.float32),
    )(x2, agg, agg, tu_pad, w_agg, bu4)


def _tc_mlp(mol, w1, b1, w2, b2):
    return pl.pallas_call(
        _tc_mlp_body,
        out_shape=jax.ShapeDtypeStruct((N_MOL, 1), jnp.float32),
    )(mol, w1, b1.reshape(1, -1), w2, b2.reshape(1, 1))


def kernel(x, edge_index, edge_attr, batch, atom_table, bond_table,
           W_init, b_init, W_upd, b_upd, W1, b1, W2, b2):
    A = atom_table.shape[1]
    B = bond_table.shape[1]

    tw_full = jnp.zeros((NTYPES, D), jnp.float32).at[:119].set(
        atom_table @ W_init[:A])
    tu_full = jnp.zeros((NTYPES, D), jnp.float32).at[:119].set(
        atom_table @ W_upd[:A])
    tw_pad = jnp.stack([tw_full[:, :H], tw_full[:, H:]])
    tu_pad = jnp.stack([tu_full[:, :H], tu_full[:, H:]])
    bondW4 = bond_table @ W_init[A:] + b_init
    bondU4 = bond_table @ W_upd[A:A + B] + b_upd
    bw4 = jnp.stack([bondW4[:, :H], bondW4[:, H:]])
    bu4 = jnp.stack([bondU4[:, :H], bondU4[:, H:]])
    w_full = W_upd[A + B:]
    w_agg = jnp.stack([w_full[:, :H], w_full[:, H:]])

    src = edge_index[0]
    dst = edge_index[1]
    pad_e = NE_PAD - N_EDGES
    e4 = jnp.concatenate([src * NBOND + edge_attr,
                          jnp.zeros((pad_e,), jnp.int32)])
    dst_p = jnp.concatenate(
        [dst, jnp.full((pad_e,), DUMMY_NODE, jnp.int32)]).reshape(-1, 128)
    batch_p = jnp.concatenate(
        [batch, jnp.full((NN_PAD - N_NODES,), DUMMY_MOL, jnp.int32)]
    ).reshape(-1, 128)
    x2 = jnp.concatenate(
        [x, jnp.zeros((NN_PAD - N_NODES,), jnp.int32)]
    ).astype(jnp.float32).reshape(NN_PAD, 1)

    zrows = jnp.zeros((NPT, H), jnp.float32)
    zmol = jnp.zeros((MOL_ROWS // NS, H), jnp.float32)

    tab = _tc_table_init(x2, tw_pad, bw4).reshape(NC * NN_PAD * NBOND, H)
    agg = _edge_pass()(e4, dst_p, tab, zrows)
    for _ in range(3):
        tab = _tc_table_pass(x2, agg, tu_pad, w_agg, bu4).reshape(
            NC * NN_PAD * NBOND, H)
        agg = _edge_pass()(e4, dst_p, tab, zrows)

    mol_halves = _mol_reduce()(agg, batch_p, zmol)
    mol = jnp.concatenate([mol_halves[0], mol_halves[1]], axis=1)
    out = _tc_mlp(mol, W1, b1, W2, b2)
    return out[:, 0]


# speedup vs baseline: 1.4362x; 1.0289x over previous
import functools

import jax
import jax.numpy as jnp
from jax import lax
from jax.experimental import pallas as pl
from jax.experimental.pallas import tpu as pltpu
from jax.experimental.pallas import tpu_sc as plsc

N_NODES = 50000
N_EDGES = 800000
N_MOL = 512
D = 64
H = 32
NC = 2
NS = 16
LANES = 16
NTYPES = 128
NBOND = 4

EPW = 50688
NE_PAD = EPW * NS
BLK = 128
NBLK = EPW // BLK
NBUF = 6
GAHEAD = 4
NGRP = NBLK // NBUF

NPT = 3200
NN_PAD = NPT * NS
DUMMY_NODE = N_NODES
MOL_ROWS = 640
DUMMY_MOL = N_MOL

_MESH = plsc.VectorSubcoreMesh(core_axis_name="c", subcore_axis_name="s")
_SC_PARAMS = pltpu.CompilerParams(use_tc_tiling_on_sc=False)


def _edge_pass_body(e4_hbm, dst2_hbm, tab_hbm, zrows_hbm, out_hbm,
                    idx_v, dst_v, rows_v, acc_sh,
                    gs0, gs1, gs2, gs3, gs4, gs5,
                    ss0, ss1, ss2, ss3, ss4, ss5):
    c = lax.axis_index("c")
    s = lax.axis_index("s")
    gs = (gs0, gs1, gs2, gs3, gs4, gs5)
    ss = (ss0, ss1, ss2, ss3, ss4, ss5)

    pltpu.sync_copy(zrows_hbm, acc_sh.at[pl.ds(s * NPT, NPT)])
    plsc.subcore_barrier()

    tab_off = c * (NBOND * NN_PAD)
    GSTEP = NBUF * BLK

    def load_group(q, g):
        e0 = s * EPW + g * GSTEP
        pltpu.sync_copy(e4_hbm.at[pl.ds(e0, GSTEP)], idx_v.at[q])

        def add_off(k, carry):
            sl = pl.ds(k * LANES, LANES)
            idx_v[q, sl] = idx_v[q, sl] + tab_off
            return carry

        lax.fori_loop(0, GSTEP // LANES, add_off, 0)
        pltpu.sync_copy(dst2_hbm.at[pl.ds(s * NBLK + g * NBUF, NBUF)],
                        dst_v.at[q])

    def fire_gather(q, brow, slot):
        pltpu.async_copy(tab_hbm.at[idx_v.at[q, pl.ds(brow * BLK, BLK)]],
                         rows_v.at[slot], gs[slot])

    def wait_gather(slot):
        pltpu.make_async_copy(tab_hbm.at[idx_v.at[0, pl.ds(0, BLK)]],
                              rows_v.at[slot], gs[slot]).wait()

    def wait_scatter(slot):
        pltpu.make_async_copy(rows_v.at[slot], acc_sh.at[dst_v.at[0, 0]],
                              ss[slot]).wait()

    load_group(0, 0)
    for b in range(GAHEAD):
        fire_gather(0, b, b)

    def group(g, carry):
        p = lax.rem(g, 2)

        @pl.when(g > 0)
        def _():
            wait_scatter(NBUF - 2)
            wait_scatter(NBUF - 1)

        @pl.when(g < NGRP - 1)
        def _():
            load_group(1 - p, g + 1)

        for b in range(NBUF):
            wait_gather(b)
            pltpu.async_copy(rows_v.at[b], acc_sh.at[dst_v.at[p, b]],
                             ss[b], add=True)
            if b >= 2:
                wait_scatter(b - 2)
            nslot = (b + GAHEAD) % NBUF
            if b < NBUF - GAHEAD:
                fire_gather(p, b + GAHEAD, nslot)
            else:
                @pl.when(g < NGRP - 1)
                def _():
                    fire_gather(1 - p, b + GAHEAD - NBUF, nslot)
        return carry

    lax.fori_loop(0, NGRP, group, 0)
    wait_scatter((NBLK - 2) % NBUF)
    wait_scatter((NBLK - 1) % NBUF)

    plsc.subcore_barrier()
    pltpu.sync_copy(acc_sh.at[pl.ds(s * NPT, NPT)],
                    out_hbm.at[pl.ds(c * NN_PAD + s * NPT, NPT)])


def _mol_body(node_hbm, batch2_hbm, zmol_hbm, out_hbm,
              bt_v, nrows_v, acc_sh, sem):
    c = lax.axis_index("c")
    s = lax.axis_index("s")
    rpt = MOL_ROWS // NS

    pltpu.sync_copy(zmol_hbm, acc_sh.at[pl.ds(s * rpt, rpt)])
    pltpu.sync_copy(batch2_hbm.at[pl.ds(s * (NPT // 128), NPT // 128)], bt_v)
    plsc.subcore_barrier()

    def block(q, carry):
        r0 = c * NN_PAD + s * NPT + q * 128
        pltpu.sync_copy(node_hbm.at[pl.ds(r0, 128)], nrows_v)
        pltpu.sync_copy(nrows_v, acc_sh.at[bt_v.at[q]], add=True)
        return carry

    lax.fori_loop(0, NPT // 128, block, 0)
    plsc.subcore_barrier()

    rows_out = N_MOL // NS
    pltpu.sync_copy(acc_sh.at[pl.ds(s * rows_out, rows_out)],
                    out_hbm.at[c, pl.ds(s * rows_out, rows_out)])


BLKN = 1024
_NBLK_N = NN_PAD // BLKN


def _tc_table_init_body(x_ref, tw_ref, tu_ref, bw_ref, out_ref, emb_ref):
    ks = lax.broadcasted_iota(jnp.int32, (BLKN, NTYPES), 1).astype(jnp.float32)
    onehot = (x_ref[...] == ks).astype(jnp.float32)
    res = jnp.dot(onehot, tw_ref[0], preferred_element_type=jnp.float32)
    out_ref[...] = jnp.maximum(res[:, None, :] + bw_ref[0][None, :, :], 0.0)
    emb_ref[...] = jnp.dot(onehot, tu_ref[0],
                           preferred_element_type=jnp.float32)


def _tc_table_pass_body(emb_ref, a0_ref, a1_ref, w_ref, bu_ref, out_ref):
    xfull = jnp.concatenate([a0_ref[...], a1_ref[...]], axis=1)
    res = (jnp.dot(xfull, w_ref[0], preferred_element_type=jnp.float32)
           + emb_ref[...])
    out_ref[...] = jnp.maximum(res[:, None, :] + bu_ref[0][None, :, :], 0.0)


def _tc_mlp_body(mol_ref, w1_ref, b1_ref, w2_ref, b2_ref, out_ref):
    h = jnp.maximum(jnp.dot(mol_ref[...], w1_ref[...],
                            preferred_element_type=jnp.float32) + b1_ref[...],
                    0.0)
    out_ref[...] = jnp.dot(h, w2_ref[...],
                           preferred_element_type=jnp.float32) + b2_ref[...]


_edge_pass = functools.partial(
    pl.kernel, _edge_pass_body,
    out_type=jax.ShapeDtypeStruct((NC * NN_PAD, H), jnp.float32),
    mesh=_MESH,
    compiler_params=_SC_PARAMS,
    scratch_types=[
        pltpu.VMEM((2, NBUF * BLK), jnp.int32),
        pltpu.VMEM((2, NBUF, BLK), jnp.int32),
        pltpu.VMEM((NBUF, BLK, H), jnp.float32),
        pltpu.VMEM_SHARED((NN_PAD, H), jnp.float32),
    ] + [pltpu.SemaphoreType.DMA] * 12,
)

_mol_reduce = functools.partial(
    pl.kernel, _mol_body,
    out_type=jax.ShapeDtypeStruct((NC, N_MOL, H), jnp.float32),
    mesh=_MESH,
    compiler_params=_SC_PARAMS,
    scratch_types=[
        pltpu.VMEM((NPT // 128, 128), jnp.int32),
        pltpu.VMEM((128, H), jnp.float32),
        pltpu.VMEM_SHARED((MOL_ROWS, H), jnp.float32),
        pltpu.SemaphoreType.DMA,
    ],
)


def _tc_table_init(x2, tw_pad, tu_pad, bw4):
    return pl.pallas_call(
        _tc_table_init_body,
        grid=(NC, _NBLK_N),
        in_specs=[
            pl.BlockSpec((BLKN, 1), lambda c, i: (i, 0)),
            pl.BlockSpec((1, NTYPES, H), lambda c, i: (c, 0, 0)),
            pl.BlockSpec((1, NTYPES, H), lambda c, i: (c, 0, 0)),
            pl.BlockSpec((1, NBOND, H), lambda c, i: (c, 0, 0)),
        ],
        out_specs=[
            pl.BlockSpec((BLKN, NBOND, H),
                         lambda c, i: (c * _NBLK_N + i, 0, 0)),
            pl.BlockSpec((BLKN, H), lambda c, i: (c * _NBLK_N + i, 0)),
        ],
        out_shape=[
            jax.ShapeDtypeStruct((NC * NN_PAD, NBOND, H), jnp.float32),
            jax.ShapeDtypeStruct((NC * NN_PAD, H), jnp.float32),
        ],
    )(x2, tw_pad, tu_pad, bw4)


def _tc_table_pass(embu, agg, w_agg, bu4):
    return pl.pallas_call(
        _tc_table_pass_body,
        grid=(NC, _NBLK_N),
        in_specs=[
            pl.BlockSpec((BLKN, H), lambda c, i: (c * _NBLK_N + i, 0)),
            pl.BlockSpec((BLKN, H), lambda c, i: (i, 0)),
            pl.BlockSpec((BLKN, H), lambda c, i: (_NBLK_N + i, 0)),
            pl.BlockSpec((1, D, H), lambda c, i: (c, 0, 0)),
            pl.BlockSpec((1, NBOND, H), lambda c, i: (c, 0, 0)),
        ],
        out_specs=pl.BlockSpec((BLKN, NBOND, H),
                               lambda c, i: (c * _NBLK_N + i, 0, 0)),
        out_shape=jax.ShapeDtypeStruct((NC * NN_PAD, NBOND, H), jnp.float32),
    )(embu, agg, agg, w_agg, bu4)


def _tc_mlp(mol, w1, b1, w2, b2):
    return pl.pallas_call(
        _tc_mlp_body,
        out_shape=jax.ShapeDtypeStruct((N_MOL, 1), jnp.float32),
    )(mol, w1, b1.reshape(1, -1), w2, b2.reshape(1, 1))


def kernel(x, edge_index, edge_attr, batch, atom_table, bond_table,
           W_init, b_init, W_upd, b_upd, W1, b1, W2, b2):
    A = atom_table.shape[1]
    B = bond_table.shape[1]

    tw_full = jnp.zeros((NTYPES, D), jnp.float32).at[:119].set(
        atom_table @ W_init[:A])
    tu_full = jnp.zeros((NTYPES, D), jnp.float32).at[:119].set(
        atom_table @ W_upd[:A])
    tw_pad = jnp.stack([tw_full[:, :H], tw_full[:, H:]])
    tu_pad = jnp.stack([tu_full[:, :H], tu_full[:, H:]])
    bondW4 = bond_table @ W_init[A:] + b_init
    bondU4 = bond_table @ W_upd[A:A + B] + b_upd
    bw4 = jnp.stack([bondW4[:, :H], bondW4[:, H:]])
    bu4 = jnp.stack([bondU4[:, :H], bondU4[:, H:]])
    w_full = W_upd[A + B:]
    w_agg = jnp.stack([w_full[:, :H], w_full[:, H:]])

    src = edge_index[0]
    dst = edge_index[1]
    pad_e = NE_PAD - N_EDGES
    e4 = jnp.concatenate([src * NBOND + edge_attr,
                          jnp.zeros((pad_e,), jnp.int32)])
    dst_p = jnp.concatenate(
        [dst, jnp.full((pad_e,), DUMMY_NODE, jnp.int32)]).reshape(-1, 128)
    batch_p = jnp.concatenate(
        [batch, jnp.full((NN_PAD - N_NODES,), DUMMY_MOL, jnp.int32)]
    ).reshape(-1, 128)
    x2 = jnp.concatenate(
        [x, jnp.zeros((NN_PAD - N_NODES,), jnp.int32)]
    ).astype(jnp.float32).reshape(NN_PAD, 1)

    zrows = jnp.zeros((NPT, H), jnp.float32)
    zmol = jnp.zeros((MOL_ROWS // NS, H), jnp.float32)

    tab3, embu = _tc_table_init(x2, tw_pad, tu_pad, bw4)
    agg = _edge_pass()(e4, dst_p, tab3.reshape(NC * NN_PAD * NBOND, H), zrows)
    for _ in range(3):
        tab = _tc_table_pass(embu, agg, w_agg, bu4).reshape(
            NC * NN_PAD * NBOND, H)
        agg = _edge_pass()(e4, dst_p, tab, zrows)

    mol_halves = _mol_reduce()(agg, batch_p, zmol)
    mol = jnp.concatenate([mol_halves[0], mol_halves[1]], axis=1)
    out = _tc_mlp(mol, W1, b1, W2, b2)
    return out[:, 0]
